# baseline (speedup 1.0000x reference)
import functools
import numpy as np
import jax
import jax.numpy as jnp
from jax import lax
from jax.experimental import pallas as pl
from jax.experimental.pallas import tpu as pltpu
from jax.experimental.pallas import tpu_sc as plsc

N_NODES = 10000
CH = 512
NP = 10240
EP = 163840
NC = 2
NW = 32
EPW = EP // NW
BE1 = 64
NBLK1 = EPW // BE1
BE3 = 256
NBLK3 = EPW // BE3
NPA = 10016
BN_C = 1.0 / np.sqrt(0.5 + 1e-3)

_mesh = plsc.VectorSubcoreMesh(core_axis_name="c", subcore_axis_name="s")


def _mm_body(x_ref, w_ref, b_ref, o_ref):
    o_ref[...] = (
        jnp.dot(x_ref[...], w_ref[...], preferred_element_type=jnp.float32)
        + b_ref[...]
    )


def _mm(x, w, b, bn=2048, bd=512):
    n, din = x.shape
    dout = w.shape[1]
    return pl.pallas_call(
        _mm_body,
        grid=(n // bn, dout // bd),
        in_specs=[
            pl.BlockSpec((bn, din), lambda i, j: (i, 0)),
            pl.BlockSpec((din, bd), lambda i, j: (0, j)),
            pl.BlockSpec((1, bd), lambda i, j: (0, j)),
        ],
        out_specs=pl.BlockSpec((bn, bd), lambda i, j: (i, j)),
        out_shape=jax.ShapeDtypeStruct((n, dout), jnp.float32),
    )(x, w, b.reshape(1, -1))


def _mlp_body(yai, aj, esi, esj, ew2, eb2, ew3, eb3, srow, sb1, sw2, sb2,
              sw3, sb3, o_ref):
    h1 = jnp.maximum(yai[...] + aj[...], 0.0)
    h2 = jnp.maximum(
        jnp.dot(h1, ew2[...], preferred_element_type=jnp.float32) + eb2[...],
        0.0)
    edge = jnp.sum(h2 * ew3[...], axis=1, keepdims=True) + eb3[...]
    g1 = jnp.maximum(esi[...] + esj[...] + edge * srow[...] + sb1[...], 0.0)
    g2 = jnp.maximum(
        jnp.dot(g1, sw2[...], preferred_element_type=jnp.float32) + sb2[...],
        0.0)
    o_ref[...] = (
        jnp.dot(g2, sw3[...], preferred_element_type=jnp.float32) + sb3[...])


def _edge_mlp(yai, aj, esi, esj, p, be=2048):
    full = lambda shape: pl.BlockSpec(shape, lambda i: tuple(0 for _ in shape))
    eb = lambda i: (i, 0)
    return pl.pallas_call(
        _mlp_body,
        grid=(EP // be,),
        in_specs=[
            pl.BlockSpec((be, 128), eb), pl.BlockSpec((be, 128), eb),
            pl.BlockSpec((be, 128), eb), pl.BlockSpec((be, 128), eb),
            full((128, 64)), full((1, 64)), full((1, 64)), full((1, 1)),
            full((1, 128)), full((1, 128)), full((128, 64)), full((1, 64)),
            full((64, 512)), full((1, 512)),
        ],
        out_specs=pl.BlockSpec((be, 512), eb),
        out_shape=jax.ShapeDtypeStruct((EP, 512), jnp.float32),
    )(yai, aj, esi, esj, p['e_w2'], p['e_b2'].reshape(1, 64),
      p['e_w3'].reshape(1, 64), p['e_b3'].reshape(1, 1),
      p['s_row'].reshape(1, 128), p['s_b1'].reshape(1, 128), p['s_w2'],
      p['s_b2'].reshape(1, 64), p['s_w3'], p['s_b3'].reshape(1, 512))


def _worker_id():
    return lax.axis_index("s") * NC + lax.axis_index("c")


@functools.partial(
    pl.kernel, mesh=_mesh,
    compiler_params=pltpu.CompilerParams(use_tc_tiling_on_sc=False),
    out_type=(
        jax.ShapeDtypeStruct((NC, NP, 136), jnp.float32),
        jax.ShapeDtypeStruct((EP, 128), jnp.float32),
        jax.ShapeDtypeStruct((EP, 128), jnp.float32),
    ),
    scratch_types=[
        pltpu.VMEM((NBLK1, BE1), jnp.int32),
        pltpu.VMEM((NBLK1, BE1), jnp.int32),
        pltpu.VMEM((BE1, 136), jnp.float32),
        pltpu.VMEM((BE1, 128), jnp.float32),
        pltpu.VMEM((BE1, 128), jnp.float32),
        pltpu.VMEM_SHARED((NP, 136), jnp.float32),
        pltpu.SemaphoreType.DMA,
        pltpu.SemaphoreType.DMA,
        pltpu.SemaphoreType.DMA,
    ],
)
def _k1(tab_baug, tab_s, idxi_hbm, idxj_hbm, zeros_hbm,
        nbr_hbm, esi_hbm, esj_hbm,
        idxi_v, idxj_v, g_b, g_si, g_sj, acc_sh, sem1, sem2, sem3):
    cid = lax.axis_index("c")
    sid = lax.axis_index("s")
    wid = sid * NC + cid
    pltpu.sync_copy(idxi_hbm.at[wid], idxi_v)
    pltpu.sync_copy(idxj_hbm.at[wid], idxj_v)

    @pl.when(sid == 0)
    def _zero():
        pltpu.sync_copy(zeros_hbm, acc_sh)

    plsc.subcore_barrier()
    base = wid * EPW

    def body(b, carry):
        cp1 = pltpu.async_copy(tab_baug.at[idxj_v.at[b]], g_b, sem1)
        cp2 = pltpu.async_copy(tab_s.at[idxi_v.at[b]], g_si, sem2)
        cp3 = pltpu.async_copy(tab_s.at[idxj_v.at[b]], g_sj, sem3)
        cp1.wait()
        cp2.wait()
        cp3.wait()
        pltpu.sync_copy(g_si, esi_hbm.at[pl.ds(base + b * BE1, BE1)])
        pltpu.sync_copy(g_sj, esj_hbm.at[pl.ds(base + b * BE1, BE1)])
        pltpu.sync_copy(g_b, acc_sh.at[idxi_v.at[b]], add=True)
        return carry

    lax.fori_loop(0, NBLK1, body, 0)
    plsc.subcore_barrier()

    @pl.when(sid == 0)
    def _out():
        pltpu.sync_copy(acc_sh, nbr_hbm.at[cid])


@functools.partial(
    pl.kernel, mesh=_mesh,
    compiler_params=pltpu.CompilerParams(use_tc_tiling_on_sc=False),
    out_type=(
        jax.ShapeDtypeStruct((EP, 128), jnp.float32),
        jax.ShapeDtypeStruct((EP, 128), jnp.float32),
    ),
    scratch_types=[
        pltpu.VMEM((NBLK3, BE3), jnp.int32),
        pltpu.VMEM((NBLK3, BE3), jnp.int32),
        pltpu.VMEM((BE3, 128), jnp.float32),
        pltpu.VMEM((BE3, 128), jnp.float32),
        pltpu.SemaphoreType.DMA,
        pltpu.SemaphoreType.DMA,
    ],
)
def _k3(tab_ya, tab_a, idxi_hbm, idxj_hbm, eyai_hbm, eaj_hbm,
        idxi_v, idxj_v, g_y, g_a, sem1, sem2):
    wid = _worker_id()
    pltpu.sync_copy(idxi_hbm.at[wid], idxi_v)
    pltpu.sync_copy(idxj_hbm.at[wid], idxj_v)
    base = wid * EPW

    def body(b, carry):
        cp1 = pltpu.async_copy(tab_ya.at[idxi_v.at[b]], g_y, sem1)
        cp2 = pltpu.async_copy(tab_a.at[idxj_v.at[b]], g_a, sem2)
        cp1.wait()
        cp2.wait()
        pltpu.sync_copy(g_y, eyai_hbm.at[pl.ds(base + b * BE3, BE3)])
        pltpu.sync_copy(g_a, eaj_hbm.at[pl.ds(base + b * BE3, BE3)])
        return carry

    lax.fori_loop(0, NBLK3, body, 0)


@functools.partial(
    pl.kernel, mesh=_mesh,
    compiler_params=pltpu.CompilerParams(use_tc_tiling_on_sc=False),
    out_type=jax.ShapeDtypeStruct((8, NC, NP, 128), jnp.float32),
    scratch_types=[
        pltpu.VMEM((NBLK1, BE1), jnp.int32),
        pltpu.VMEM((NBLK1, BE1), jnp.int32),
        pltpu.VMEM((BE1, 64), jnp.float32),
        pltpu.VMEM((BE1, 64), jnp.float32),
        pltpu.VMEM((BE1, 128), jnp.float32),
        pltpu.VMEM((BE1, 128), jnp.float32),
        pltpu.VMEM((BE1, 64), jnp.float32),
        pltpu.VMEM((BE1, 64), jnp.float32),
        pltpu.VMEM((BE1, 64), jnp.float32),
        pltpu.VMEM_SHARED((NPA, 64), jnp.float32),
        pltpu.VMEM_SHARED((NPA, 64), jnp.float32),
        pltpu.SemaphoreType.DMA,
        pltpu.SemaphoreType.DMA,
        pltpu.SemaphoreType.DMA,
        pltpu.SemaphoreType.DMA,
        pltpu.SemaphoreType.DMA,
        pltpu.SemaphoreType.DMA,
    ],
)
def _k5(tab_q, tab_kv, sz_hbm, idxi_hbm, idxj_hbm, zeros64_hbm, nd_hbm,
        idxi_v, idxj_v, q0, q1, kv0, kv1, sz0, sz1, m_v,
        acc_e, acc_m, sq0, sq1, sk0, sk1, sz0s, sz1s):
    cid = lax.axis_index("c")
    sid = lax.axis_index("s")
    wid = sid * NC + cid
    pltpu.sync_copy(idxi_hbm.at[wid], idxi_v)
    pltpu.sync_copy(idxj_hbm.at[wid], idxj_v)
    base = wid * EPW
    rings = ((q0, kv0, sz0, sq0, sk0, sz0s), (q1, kv1, sz1, sq1, sk1, sz1s))

    for c in range(8):
        def _issue(b, ring):
            qb, kvb, szb, s1, s2, s3 = ring
            pltpu.async_copy(tab_q.at[c].at[idxi_v.at[b]], qb, s1)
            pltpu.async_copy(tab_kv.at[c].at[idxj_v.at[b]], kvb, s2)
            pltpu.async_copy(
                sz_hbm.at[pl.ds(base + b * BE1, BE1), pl.ds(c * 64, 64)],
                szb, s3)

        def _wait(b, ring):
            qb, kvb, szb, s1, s2, s3 = ring
            pltpu.make_async_copy(tab_q.at[c].at[idxi_v.at[b]], qb, s1).wait()
            pltpu.make_async_copy(
                tab_kv.at[c].at[idxj_v.at[b]], kvb, s2).wait()
            pltpu.make_async_copy(
                sz_hbm.at[pl.ds(base + b * BE1, BE1), pl.ds(c * 64, 64)],
                szb, s3).wait()

        def _compute_scatter(b, ring):
            qb, kvb, szb = ring[0], ring[1], ring[2]

            def ebody(t2, c2):
                for dt in range(2):
                    t = t2 * 2 + dt
                    for r in range(4):
                        sl = pl.ds(r * 16, 16)
                        sl2 = pl.ds(64 + r * 16, 16)
                        e = jnp.exp(qb[t, sl] * kvb[t, sl])
                        qb[t, sl] = e
                        m_v[t, sl] = e * kvb[t, sl2] * szb[t, sl]
                return c2

            lax.fori_loop(0, BE1 // 2, ebody, 0)
            pltpu.sync_copy(qb, acc_e.at[idxi_v.at[b]], add=True)
            pltpu.sync_copy(m_v, acc_m.at[idxi_v.at[b]], add=True)

        @pl.when(sid == 0)
        def _zero():
            pltpu.sync_copy(zeros64_hbm, acc_e)
            pltpu.sync_copy(zeros64_hbm, acc_m)

        plsc.subcore_barrier()
        _issue(0, rings[0])

        def body2(h, carry):
            b0 = 2 * h
            b1 = 2 * h + 1
            _issue(b1, rings[1])
            _wait(b0, rings[0])
            _compute_scatter(b0, rings[0])

            @pl.when(b1 + 1 < NBLK1)
            def _next():
                _issue(b1 + 1, rings[0])

            _wait(b1, rings[1])
            _compute_scatter(b1, rings[1])
            return carry

        lax.fori_loop(0, NBLK1 // 2, body2, 0)
        plsc.subcore_barrier()

        @pl.when(sid == 0)
        def _out():
            pltpu.sync_copy(
                acc_e, nd_hbm.at[c].at[cid].at[pl.ds(0, NPA), pl.ds(0, 64)])
            pltpu.sync_copy(
                acc_m, nd_hbm.at[c].at[cid].at[pl.ds(0, NPA), pl.ds(64, 64)])


def _k2_body(ta, tb, nbr, eb1, ya_ref, invd_ref):
    nd = nbr[...]
    nb = nd[0, :, :128] + nd[1, :, :128]
    cnt = nd[0, :, 128] + nd[1, :, 128]
    maskc = (cnt > 0).astype(jnp.float32)
    invd = 1.0 / jnp.maximum(cnt, 1.0)
    ya_ref[...] = (ta[...] + 0.5 * maskc[:, None] * tb[...]
                   + 0.5 * invd[:, None] * nb + eb1[...])
    invd_ref[...] = jnp.broadcast_to(invd[:, None], invd_ref.shape)


def _node_combine(tab_a, tab_b, nbracc, e_b1, bn=2048):
    return pl.pallas_call(
        _k2_body,
        grid=(NP // bn,),
        in_specs=[
            pl.BlockSpec((bn, 128), lambda i: (i, 0)),
            pl.BlockSpec((bn, 128), lambda i: (i, 0)),
            pl.BlockSpec((NC, bn, 136), lambda i: (0, i, 0)),
            pl.BlockSpec((1, 128), lambda i: (0, 0)),
        ],
        out_specs=[
            pl.BlockSpec((bn, 128), lambda i: (i, 0)),
            pl.BlockSpec((bn, 64), lambda i: (i, 0)),
        ],
        out_shape=[
            jax.ShapeDtypeStruct((NP, 128), jnp.float32),
            jax.ShapeDtypeStruct((NP, 64), jnp.float32),
        ],
    )(tab_a, tab_b, nbracc, e_b1.reshape(1, 128))


def _k6_body(xw, nd_ref, invd, o_ref):
    nd = nd_ref[...]
    e = nd[:, 0, :, :64] + nd[:, 1, :, :64]
    m = nd[:, 0, :, 64:] + nd[:, 1, :, 64:]
    agg = m / (e + 1e-9)
    agg128 = jnp.concatenate([agg[0], agg[1]], axis=1)
    iv = invd[...]
    iv128 = jnp.concatenate([iv, iv], axis=1)
    o_ref[...] = jnp.maximum(
        jnp.maximum(xw[...], 0.0) + agg128 * iv128 * BN_C, 0.0)


def _finalize(xw, numden, invd, bn=2048):
    return pl.pallas_call(
        _k6_body,
        grid=(NP // bn, 4),
        in_specs=[
            pl.BlockSpec((bn, 128), lambda i, c: (i, c)),
            pl.BlockSpec((2, NC, bn, 128), lambda i, c: (c, 0, i, 0)),
            pl.BlockSpec((bn, 64), lambda i, c: (i, 0)),
        ],
        out_specs=pl.BlockSpec((bn, 128), lambda i, c: (i, c)),
        out_shape=jax.ShapeDtypeStruct((NP, 512), jnp.float32),
    )(xw, numden, invd)


def _prep_layer_params(p, din):
    inv = 1.0 / np.sqrt(float(CH))
    wcat1 = jnp.concatenate([
        0.5 * p['e_w1'][:din], p['e_w1'][din:], 0.5 * p['s_w1'][:din],
        jnp.zeros((din, 128), jnp.float32),
    ], axis=1)
    bcat1 = jnp.zeros((512,), jnp.float32)
    wcat2 = jnp.concatenate([
        p['q_w'] * inv, p['k_w'], p['v_w'], p['wint_w'],
    ], axis=1)
    bcat2 = jnp.concatenate([
        p['q_b'] * inv, p['k_b'], p['v_b'], p['wint_b'],
    ])
    return {
        'wcat1': wcat1, 'bcat1': bcat1, 'wcat2': wcat2, 'bcat2': bcat2,
        'e_b1': p['e_b1'], 'e_w2': p['e_w2'], 'e_b2': p['e_b2'],
        'e_w3': p['e_w3'][:, 0], 'e_b3': p['e_b3'],
        's_row': p['s_w1'][din], 's_b1': p['s_b1'],
        's_w2': p['s_w2'], 's_b2': p['s_b2'],
        's_w3': p['s_w3'], 's_b3': p['s_b3'],
    }


def _layer(xpad, pp, idx3a, idx3b, zeros136, zeros64):
    idx_i3a, idx_j3a = idx3a
    idx_i3b, idx_j3b = idx3b
    proj1 = _mm(xpad, pp['wcat1'], pp['bcat1'])
    tab_a = proj1[:, 0:128]
    tab_b = proj1[:, 128:256]
    tab_s = proj1[:, 256:384]

    tab_baug = jnp.concatenate([
        tab_b,
        jnp.ones((NP, 1), jnp.float32),
        jnp.zeros((NP, 7), jnp.float32),
    ], axis=1)

    nbracc, esi, esj = _k1(tab_baug, tab_s, idx_i3a, idx_j3a, zeros136)
    proj2 = _mm(xpad, pp['wcat2'], pp['bcat2'])
    xq = proj2[:, 0:512]
    xk = proj2[:, 512:1024]
    xv = proj2[:, 1024:1536]
    xw = proj2[:, 1536:2048]
    ya, invd = _node_combine(tab_a, tab_b, nbracc, pp['e_b1'])
    eyai, eaj = _k3(ya, tab_a, idx_i3b, idx_j3b)
    sz = _edge_mlp(eyai, eaj, esi, esj, pp)

    tab_q = xq.reshape(NP, 8, 64).transpose(1, 0, 2)
    tab_kv = jnp.concatenate([
        xk.reshape(NP, 8, 64).transpose(1, 0, 2),
        xv.reshape(NP, 8, 64).transpose(1, 0, 2),
    ], axis=2)
    numden = _k5(tab_q, tab_kv, sz, idx_i3a, idx_j3a, zeros64)
    return _finalize(xw, numden, invd)


def kernel(x, edge_index, params):
    npad = EP - edge_index.shape[1]
    idx_i = jnp.concatenate(
        [edge_index[0], jnp.full((npad,), N_NODES, jnp.int32)])
    idx_j = jnp.concatenate(
        [edge_index[1], jnp.full((npad,), N_NODES, jnp.int32)])
    idx_i3a = idx_i.reshape(NW, NBLK1, BE1)
    idx_j3a = idx_j.reshape(NW, NBLK1, BE1)
    idx_i3b = idx_i.reshape(NW, NBLK3, BE3)
    idx_j3b = idx_j.reshape(NW, NBLK3, BE3)
    xpad = jnp.zeros((NP, x.shape[1]), jnp.float32).at[:N_NODES].set(x)

    zeros136 = jnp.zeros((NP, 136), jnp.float32)
    zeros64 = jnp.zeros((NPA, 64), jnp.float32)

    p1 = _prep_layer_params(params['layer1'], 256)
    p2 = _prep_layer_params(params['layer2'], 512)
    ia = (idx_i3a, idx_j3a)
    ib = (idx_i3b, idx_j3b)
    h = _layer(xpad, p1, ia, ib, zeros136, zeros64)
    h = _layer(h, p2, ia, ib, zeros136, zeros64)
    return h[:N_NODES]


# speedup vs baseline: 1.0137x; 1.0137x over previous
import functools
import numpy as np
import jax
import jax.numpy as jnp
from jax import lax
from jax.experimental import pallas as pl
from jax.experimental.pallas import tpu as pltpu
from jax.experimental.pallas import tpu_sc as plsc

N_NODES = 10000
CH = 512
NP = 10240
EP = 163840
NC = 2
NW = 32
EPW = EP // NW
BE1 = 64
NBLK1 = EPW // BE1
BE3 = 128
NBLK3 = EPW // BE3
BEK = 32
NBLKK = EPW // BEK
NPA = 10016
BN_C = 1.0 / np.sqrt(0.5 + 1e-3)

_mesh = plsc.VectorSubcoreMesh(core_axis_name="c", subcore_axis_name="s")


def _mm_body(x_ref, w_ref, b_ref, o_ref):
    o_ref[...] = (
        jnp.dot(x_ref[...], w_ref[...], preferred_element_type=jnp.float32)
        + b_ref[...]
    )


def _mm(x, w, b, bn=2048, bd=512):
    n, din = x.shape
    dout = w.shape[1]
    return pl.pallas_call(
        _mm_body,
        grid=(n // bn, dout // bd),
        in_specs=[
            pl.BlockSpec((bn, din), lambda i, j: (i, 0)),
            pl.BlockSpec((din, bd), lambda i, j: (0, j)),
            pl.BlockSpec((1, bd), lambda i, j: (0, j)),
        ],
        out_specs=pl.BlockSpec((bn, bd), lambda i, j: (i, j)),
        out_shape=jax.ShapeDtypeStruct((n, dout), jnp.float32),
    )(x, w, b.reshape(1, -1))


def _mlp_body(yai, aj, esi, esj, ew2, eb2, ew3, eb3, srow, sb1, sw2, sb2,
              sw3, sb3, o_ref):
    h1 = jnp.maximum(yai[...] + aj[...], 0.0)
    h2 = jnp.maximum(
        jnp.dot(h1, ew2[...], preferred_element_type=jnp.float32) + eb2[...],
        0.0)
    edge = jnp.sum(h2 * ew3[...], axis=1, keepdims=True) + eb3[...]
    g1 = jnp.maximum(esi[...] + esj[...] + edge * srow[...] + sb1[...], 0.0)
    g2 = jnp.maximum(
        jnp.dot(g1, sw2[...], preferred_element_type=jnp.float32) + sb2[...],
        0.0)
    o_ref[...] = (
        jnp.dot(g2, sw3[...], preferred_element_type=jnp.float32) + sb3[...])


def _edge_mlp(yai, aj, esi, esj, p, be=2048):
    full = lambda shape: pl.BlockSpec(shape, lambda i: tuple(0 for _ in shape))
    eb = lambda i: (i, 0)
    return pl.pallas_call(
        _mlp_body,
        grid=(EP // be,),
        in_specs=[
            pl.BlockSpec((be, 128), eb), pl.BlockSpec((be, 128), eb),
            pl.BlockSpec((be, 128), eb), pl.BlockSpec((be, 128), eb),
            full((128, 64)), full((1, 64)), full((1, 64)), full((1, 1)),
            full((1, 128)), full((1, 128)), full((128, 64)), full((1, 64)),
            full((64, 512)), full((1, 512)),
        ],
        out_specs=pl.BlockSpec((be, 512), eb),
        out_shape=jax.ShapeDtypeStruct((EP, 512), jnp.float32),
    )(yai, aj, esi, esj, p['e_w2'], p['e_b2'].reshape(1, 64),
      p['e_w3'].reshape(1, 64), p['e_b3'].reshape(1, 1),
      p['s_row'].reshape(1, 128), p['s_b1'].reshape(1, 128), p['s_w2'],
      p['s_b2'].reshape(1, 64), p['s_w3'], p['s_b3'].reshape(1, 512))


def _worker_id():
    return lax.axis_index("s") * NC + lax.axis_index("c")


@functools.partial(
    pl.kernel, mesh=_mesh,
    compiler_params=pltpu.CompilerParams(use_tc_tiling_on_sc=False),
    out_type=(
        jax.ShapeDtypeStruct((NC, NP, 136), jnp.float32),
        jax.ShapeDtypeStruct((EP, 128), jnp.float32),
        jax.ShapeDtypeStruct((EP, 128), jnp.float32),
        jax.ShapeDtypeStruct((EP, 128), jnp.float32),
    ),
    scratch_types=[
        pltpu.VMEM((NBLKK, BEK), jnp.int32),
        pltpu.VMEM((NBLKK, BEK), jnp.int32),
        pltpu.VMEM((BEK, 136), jnp.float32),
        pltpu.VMEM((BEK, 136), jnp.float32),
        pltpu.VMEM((BEK, 128), jnp.float32),
        pltpu.VMEM((BEK, 128), jnp.float32),
        pltpu.VMEM((BEK, 128), jnp.float32),
        pltpu.VMEM((BEK, 128), jnp.float32),
        pltpu.VMEM((BEK, 128), jnp.float32),
        pltpu.VMEM((BEK, 128), jnp.float32),
        pltpu.VMEM_SHARED((NPA, 136), jnp.float32),
        pltpu.SemaphoreType.DMA,
        pltpu.SemaphoreType.DMA,
        pltpu.SemaphoreType.DMA,
        pltpu.SemaphoreType.DMA,
        pltpu.SemaphoreType.DMA,
        pltpu.SemaphoreType.DMA,
        pltpu.SemaphoreType.DMA,
        pltpu.SemaphoreType.DMA,
    ],
)
def _k1(tab_baug, tab_s, tab_a, idxi_hbm, idxj_hbm, zeros_hbm,
        nbr_hbm, esi_hbm, esj_hbm, eaj_hbm,
        idxi_v, idxj_v, b0, b1, si0, si1, sj0, sj1, a0, a1, acc_sh,
        sb0, sb1, ssi0, ssi1, ssj0, ssj1, sa0, sa1):
    cid = lax.axis_index("c")
    sid = lax.axis_index("s")
    wid = sid * NC + cid
    pltpu.sync_copy(idxi_hbm.at[wid], idxi_v)
    pltpu.sync_copy(idxj_hbm.at[wid], idxj_v)

    @pl.when(sid == 0)
    def _zero():
        pltpu.sync_copy(zeros_hbm.at[pl.ds(0, NPA)], acc_sh)

    plsc.subcore_barrier()
    base = wid * EPW
    rings = ((b0, si0, sj0, a0, sb0, ssi0, ssj0, sa0),
             (b1, si1, sj1, a1, sb1, ssi1, ssj1, sa1))

    def _gissue(b, ring):
        gb, gsi, gsj, ga, s1, s2, s3, s4 = ring
        pltpu.async_copy(tab_baug.at[idxj_v.at[b]], gb, s1)
        pltpu.async_copy(tab_s.at[idxi_v.at[b]], gsi, s2)
        pltpu.async_copy(tab_s.at[idxj_v.at[b]], gsj, s3)
        pltpu.async_copy(tab_a.at[idxj_v.at[b]], ga, s4)

    def _gwait(b, ring):
        gb, gsi, gsj, ga, s1, s2, s3, s4 = ring
        pltpu.make_async_copy(tab_baug.at[idxj_v.at[b]], gb, s1).wait()
        pltpu.make_async_copy(tab_s.at[idxi_v.at[b]], gsi, s2).wait()
        pltpu.make_async_copy(tab_s.at[idxj_v.at[b]], gsj, s3).wait()
        pltpu.make_async_copy(tab_a.at[idxj_v.at[b]], ga, s4).wait()

    _gissue(0, rings[0])
    _gissue(1, rings[1])

    def body2(h, carry):
        for par in (0, 1):
            b = 2 * h + par
            ring = rings[par]
            gb, gsi, gsj, ga = ring[0], ring[1], ring[2], ring[3]
            _gwait(b, ring)
            pltpu.sync_copy(gsi, esi_hbm.at[pl.ds(base + b * BEK, BEK)])
            pltpu.sync_copy(gsj, esj_hbm.at[pl.ds(base + b * BEK, BEK)])
            pltpu.sync_copy(ga, eaj_hbm.at[pl.ds(base + b * BEK, BEK)])
            pltpu.sync_copy(gb, acc_sh.at[idxi_v.at[b]], add=True)

            @pl.when(b + 2 < NBLKK)
            def _nxt():
                _gissue(b + 2, ring)
        return carry

    lax.fori_loop(0, NBLKK // 2, body2, 0)
    plsc.subcore_barrier()

    @pl.when(sid == 0)
    def _out():
        pltpu.sync_copy(
            acc_sh, nbr_hbm.at[cid].at[pl.ds(0, NPA), pl.ds(0, 136)])


@functools.partial(
    pl.kernel, mesh=_mesh,
    compiler_params=pltpu.CompilerParams(use_tc_tiling_on_sc=False),
    out_type=jax.ShapeDtypeStruct((EP, 128), jnp.float32),
    scratch_types=[
        pltpu.VMEM((NBLK3, BE3), jnp.int32),
        pltpu.VMEM((BE3, 128), jnp.float32),
        pltpu.VMEM((BE3, 128), jnp.float32),
        pltpu.SemaphoreType.DMA,
        pltpu.SemaphoreType.DMA,
    ],
)
def _k3(tab_ya, idxi_hbm, eyai_hbm, idxi_v, g0, g1, s0, s1):
    wid = _worker_id()
    pltpu.sync_copy(idxi_hbm.at[wid], idxi_v)
    base = wid * EPW
    rings = ((g0, s0), (g1, s1))

    def _gissue(b, ring):
        pltpu.async_copy(tab_ya.at[idxi_v.at[b]], ring[0], ring[1])

    def _gwait(b, ring):
        pltpu.make_async_copy(tab_ya.at[idxi_v.at[b]], ring[0], ring[1]).wait()

    _gissue(0, rings[0])
    _gissue(1, rings[1])

    def body2(h, carry):
        for par in (0, 1):
            b = 2 * h + par
            ring = rings[par]
            _gwait(b, ring)
            pltpu.sync_copy(ring[0], eyai_hbm.at[pl.ds(base + b * BE3, BE3)])

            @pl.when(b + 2 < NBLK3)
            def _nxt():
                _gissue(b + 2, ring)
        return carry

    lax.fori_loop(0, NBLK3 // 2, body2, 0)


@functools.partial(
    pl.kernel, mesh=_mesh,
    compiler_params=pltpu.CompilerParams(use_tc_tiling_on_sc=False),
    out_type=jax.ShapeDtypeStruct((8, NC, NP, 128), jnp.float32),
    scratch_types=[
        pltpu.VMEM((NBLK1, BE1), jnp.int32),
        pltpu.VMEM((NBLK1, BE1), jnp.int32),
        pltpu.VMEM((BE1, 64), jnp.float32),
        pltpu.VMEM((BE1, 64), jnp.float32),
        pltpu.VMEM((BE1, 128), jnp.float32),
        pltpu.VMEM((BE1, 128), jnp.float32),
        pltpu.VMEM((BE1, 64), jnp.float32),
        pltpu.VMEM((BE1, 64), jnp.float32),
        pltpu.VMEM((BE1, 64), jnp.float32),
        pltpu.VMEM_SHARED((NPA, 64), jnp.float32),
        pltpu.VMEM_SHARED((NPA, 64), jnp.float32),
        pltpu.SemaphoreType.DMA,
        pltpu.SemaphoreType.DMA,
        pltpu.SemaphoreType.DMA,
        pltpu.SemaphoreType.DMA,
        pltpu.SemaphoreType.DMA,
        pltpu.SemaphoreType.DMA,
    ],
)
def _k5(tab_q, tab_kv, sz_hbm, idxi_hbm, idxj_hbm, zeros64_hbm, nd_hbm,
        idxi_v, idxj_v, q0, q1, kv0, kv1, sz0, sz1, m_v,
        acc_e, acc_m, sq0, sq1, sk0, sk1, sz0s, sz1s):
    cid = lax.axis_index("c")
    sid = lax.axis_index("s")
    wid = sid * NC + cid
    pltpu.sync_copy(idxi_hbm.at[wid], idxi_v)
    pltpu.sync_copy(idxj_hbm.at[wid], idxj_v)
    base = wid * EPW
    rings = ((q0, kv0, sz0, sq0, sk0, sz0s), (q1, kv1, sz1, sq1, sk1, sz1s))

    for c in range(8):
        def _issue(b, ring):
            qb, kvb, szb, s1, s2, s3 = ring
            pltpu.async_copy(tab_q.at[c].at[idxi_v.at[b]], qb, s1)
            pltpu.async_copy(tab_kv.at[c].at[idxj_v.at[b]], kvb, s2)
            pltpu.async_copy(
                sz_hbm.at[pl.ds(base + b * BE1, BE1), pl.ds(c * 64, 64)],
                szb, s3)

        def _wait(b, ring):
            qb, kvb, szb, s1, s2, s3 = ring
            pltpu.make_async_copy(tab_q.at[c].at[idxi_v.at[b]], qb, s1).wait()
            pltpu.make_async_copy(
                tab_kv.at[c].at[idxj_v.at[b]], kvb, s2).wait()
            pltpu.make_async_copy(
                sz_hbm.at[pl.ds(base + b * BE1, BE1), pl.ds(c * 64, 64)],
                szb, s3).wait()

        def _compute_scatter(b, ring):
            qb, kvb, szb = ring[0], ring[1], ring[2]

            def ebody(t2, c2):
                for dt in range(2):
                    t = t2 * 2 + dt
                    for r in range(4):
                        sl = pl.ds(r * 16, 16)
                        sl2 = pl.ds(64 + r * 16, 16)
                        e = jnp.exp(qb[t, sl] * kvb[t, sl])
                        qb[t, sl] = e
                        m_v[t, sl] = e * kvb[t, sl2] * szb[t, sl]
                return c2

            lax.fori_loop(0, BE1 // 2, ebody, 0)
            pltpu.sync_copy(qb, acc_e.at[idxi_v.at[b]], add=True)
            pltpu.sync_copy(m_v, acc_m.at[idxi_v.at[b]], add=True)

        @pl.when(sid == 0)
        def _zero():
            pltpu.sync_copy(zeros64_hbm, acc_e)
            pltpu.sync_copy(zeros64_hbm, acc_m)

        plsc.subcore_barrier()
        _issue(0, rings[0])

        def body2(h, carry):
            b0 = 2 * h
            b1 = 2 * h + 1
            _issue(b1, rings[1])
            _wait(b0, rings[0])
            _compute_scatter(b0, rings[0])

            @pl.when(b1 + 1 < NBLK1)
            def _next():
                _issue(b1 + 1, rings[0])

            _wait(b1, rings[1])
            _compute_scatter(b1, rings[1])
            return carry

        lax.fori_loop(0, NBLK1 // 2, body2, 0)
        plsc.subcore_barrier()

        @pl.when(sid == 0)
        def _out():
            pltpu.sync_copy(
                acc_e, nd_hbm.at[c].at[cid].at[pl.ds(0, NPA), pl.ds(0, 64)])
            pltpu.sync_copy(
                acc_m, nd_hbm.at[c].at[cid].at[pl.ds(0, NPA), pl.ds(64, 64)])


def _k2_body(ta, tb, nbr, eb1, ya_ref, invd_ref):
    nd = nbr[...]
    nb = nd[0, :, :128] + nd[1, :, :128]
    cnt = nd[0, :, 128] + nd[1, :, 128]
    maskc = (cnt > 0).astype(jnp.float32)
    invd = 1.0 / jnp.maximum(cnt, 1.0)
    ya_ref[...] = (ta[...] + 0.5 * maskc[:, None] * tb[...]
                   + 0.5 * invd[:, None] * nb + eb1[...])
    invd_ref[...] = jnp.broadcast_to(invd[:, None], invd_ref.shape)


def _node_combine(tab_a, tab_b, nbracc, e_b1, bn=2048):
    return pl.pallas_call(
        _k2_body,
        grid=(NP // bn,),
        in_specs=[
            pl.BlockSpec((bn, 128), lambda i: (i, 0)),
            pl.BlockSpec((bn, 128), lambda i: (i, 0)),
            pl.BlockSpec((NC, bn, 136), lambda i: (0, i, 0)),
            pl.BlockSpec((1, 128), lambda i: (0, 0)),
        ],
        out_specs=[
            pl.BlockSpec((bn, 128), lambda i: (i, 0)),
            pl.BlockSpec((bn, 64), lambda i: (i, 0)),
        ],
        out_shape=[
            jax.ShapeDtypeStruct((NP, 128), jnp.float32),
            jax.ShapeDtypeStruct((NP, 64), jnp.float32),
        ],
    )(tab_a, tab_b, nbracc, e_b1.reshape(1, 128))


def _k6_body(xw, nd_ref, invd, o_ref):
    nd = nd_ref[...]
    e = nd[:, 0, :, :64] + nd[:, 1, :, :64]
    m = nd[:, 0, :, 64:] + nd[:, 1, :, 64:]
    agg = m / (e + 1e-9)
    agg128 = jnp.concatenate([agg[0], agg[1]], axis=1)
    iv = invd[...]
    iv128 = jnp.concatenate([iv, iv], axis=1)
    o_ref[...] = jnp.maximum(
        jnp.maximum(xw[...], 0.0) + agg128 * iv128 * BN_C, 0.0)


def _finalize(xw, numden, invd, bn=2048):
    return pl.pallas_call(
        _k6_body,
        grid=(NP // bn, 4),
        in_specs=[
            pl.BlockSpec((bn, 128), lambda i, c: (i, c)),
            pl.BlockSpec((2, NC, bn, 128), lambda i, c: (c, 0, i, 0)),
            pl.BlockSpec((bn, 64), lambda i, c: (i, 0)),
        ],
        out_specs=pl.BlockSpec((bn, 128), lambda i, c: (i, c)),
        out_shape=jax.ShapeDtypeStruct((NP, 512), jnp.float32),
    )(xw, numden, invd)


def _prep_layer_params(p, din):
    inv = 1.0 / np.sqrt(float(CH))
    wcat = jnp.concatenate([
        p['q_w'] * inv, p['k_w'], p['v_w'], p['wint_w'],
        0.5 * p['e_w1'][:din], p['e_w1'][din:], 0.5 * p['s_w1'][:din],
        jnp.zeros((din, 128), jnp.float32),
    ], axis=1)
    bcat = jnp.concatenate([
        p['q_b'] * inv, p['k_b'], p['v_b'], p['wint_b'],
        jnp.zeros((384 + 128,), jnp.float32),
    ])
    return {
        'wcat': wcat, 'bcat': bcat,
        'e_b1': p['e_b1'], 'e_w2': p['e_w2'], 'e_b2': p['e_b2'],
        'e_w3': p['e_w3'][:, 0], 'e_b3': p['e_b3'],
        's_row': p['s_w1'][din], 's_b1': p['s_b1'],
        's_w2': p['s_w2'], 's_b2': p['s_b2'],
        's_w3': p['s_w3'], 's_b3': p['s_b3'],
    }


def _layer(xpad, pp, idx3a, idx3b, idx3k, zeros136, zeros64):
    idx_i3a, idx_j3a = idx3a
    idx_i3b, idx_j3b = idx3b
    idx_i3k, idx_j3k = idx3k
    proj = _mm(xpad, pp['wcat'], pp['bcat'])
    xq = proj[:, 0:512]
    xk = proj[:, 512:1024]
    xv = proj[:, 1024:1536]
    xw = proj[:, 1536:2048]
    tab_a = proj[:, 2048:2176]
    tab_b = proj[:, 2176:2304]
    tab_s = proj[:, 2304:2432]

    tab_baug = jnp.concatenate([
        tab_b,
        jnp.ones((NP, 1), jnp.float32),
        jnp.zeros((NP, 7), jnp.float32),
    ], axis=1)

    nbracc, esi, esj, eaj = _k1(tab_baug, tab_s, tab_a, idx_i3k, idx_j3k,
                                zeros136)
    ya, invd = _node_combine(tab_a, tab_b, nbracc, pp['e_b1'])
    eyai = _k3(ya, idx_i3b)
    sz = _edge_mlp(eyai, eaj, esi, esj, pp)

    tab_q = xq.reshape(NP, 8, 64).transpose(1, 0, 2)
    tab_kv = jnp.concatenate([
        xk.reshape(NP, 8, 64).transpose(1, 0, 2),
        xv.reshape(NP, 8, 64).transpose(1, 0, 2),
    ], axis=2)
    numden = _k5(tab_q, tab_kv, sz, idx_i3a, idx_j3a, zeros64)
    return _finalize(xw, numden, invd)


def kernel(x, edge_index, params):
    npad = EP - edge_index.shape[1]
    idx_i = jnp.concatenate(
        [edge_index[0], jnp.full((npad,), N_NODES, jnp.int32)])
    idx_j = jnp.concatenate(
        [edge_index[1], jnp.full((npad,), N_NODES, jnp.int32)])
    idx_i3a = idx_i.reshape(NW, NBLK1, BE1)
    idx_j3a = idx_j.reshape(NW, NBLK1, BE1)
    idx_i3b = idx_i.reshape(NW, NBLK3, BE3)
    idx_j3b = idx_j.reshape(NW, NBLK3, BE3)
    idx_i3k = idx_i.reshape(NW, NBLKK, BEK)
    idx_j3k = idx_j.reshape(NW, NBLKK, BEK)
    xpad = jnp.zeros((NP, x.shape[1]), jnp.float32).at[:N_NODES].set(x)

    zeros136 = jnp.zeros((NP, 136), jnp.float32)
    zeros64 = jnp.zeros((NPA, 64), jnp.float32)

    p1 = _prep_layer_params(params['layer1'], 256)
    p2 = _prep_layer_params(params['layer2'], 512)
    ia = (idx_i3a, idx_j3a)
    ib = (idx_i3b, idx_j3b)
    ik = (idx_i3k, idx_j3k)
    h = _layer(xpad, p1, ia, ib, ik, zeros136, zeros64)
    h = _layer(h, p2, ia, ib, ik, zeros136, zeros64)
    return h[:N_NODES]


# speedup vs baseline: 1.0594x; 1.0451x over previous
import functools
import numpy as np
import jax
import jax.numpy as jnp
from jax import lax
from jax.experimental import pallas as pl
from jax.experimental.pallas import tpu as pltpu
from jax.experimental.pallas import tpu_sc as plsc

N_NODES = 10000
CH = 512
NP = 10240
EP = 163840
NC = 2
NW = 32
EPW = EP // NW
BE1 = 64
NBLK1 = EPW // BE1
BE3 = 256
NBLK3 = EPW // BE3
NPA = 10016
BN_C = 1.0 / np.sqrt(0.5 + 1e-3)

_mesh = plsc.VectorSubcoreMesh(core_axis_name="c", subcore_axis_name="s")


def _mm_body(x_ref, w_ref, b_ref, o_ref):
    o_ref[...] = (
        jnp.dot(x_ref[...], w_ref[...], preferred_element_type=jnp.float32)
        + b_ref[...]
    )


def _mm(x, w, b, bn=2048, bd=512):
    n, din = x.shape
    dout = w.shape[1]
    return pl.pallas_call(
        _mm_body,
        grid=(n // bn, dout // bd),
        in_specs=[
            pl.BlockSpec((bn, din), lambda i, j: (i, 0)),
            pl.BlockSpec((din, bd), lambda i, j: (0, j)),
            pl.BlockSpec((1, bd), lambda i, j: (0, j)),
        ],
        out_specs=pl.BlockSpec((bn, bd), lambda i, j: (i, j)),
        out_shape=jax.ShapeDtypeStruct((n, dout), jnp.float32),
    )(x, w, b.reshape(1, -1))


def _mm_split_body(nsl, x_ref, w_ref, b_ref, o_ref):
    acc = (jnp.dot(x_ref[...], w_ref[...], preferred_element_type=jnp.float32)
           + b_ref[...])
    sw = o_ref.shape[2]
    for k in range(nsl):
        o_ref[k] = acc[:, k * sw:(k + 1) * sw]


def _mm_split(x, w, b, sw, bn=2048, bd=256):
    n, din = x.shape
    dout = w.shape[1]
    nsl = bd // sw
    return pl.pallas_call(
        functools.partial(_mm_split_body, nsl),
        grid=(n // bn, dout // bd),
        in_specs=[
            pl.BlockSpec((bn, din), lambda i, j: (i, 0)),
            pl.BlockSpec((din, bd), lambda i, j: (0, j)),
            pl.BlockSpec((1, bd), lambda i, j: (0, j)),
        ],
        out_specs=pl.BlockSpec((nsl, bn, sw), lambda i, j: (j, i, 0)),
        out_shape=jax.ShapeDtypeStruct((dout // sw, n, sw), jnp.float32),
    )(x, w, b.reshape(1, -1))


def _mlp_body(yai, aj, esi, esj, ew2, eb2, ew3, eb3, srow, sb1, sw2, sb2,
              sw3, sb3, o_ref):
    h1 = jnp.maximum(yai[...] + aj[...], 0.0)
    h2 = jnp.maximum(
        jnp.dot(h1, ew2[...], preferred_element_type=jnp.float32) + eb2[...],
        0.0)
    edge = jnp.sum(h2 * ew3[...], axis=1, keepdims=True) + eb3[...]
    g1 = jnp.maximum(esi[...] + esj[...] + edge * srow[...] + sb1[...], 0.0)
    g2 = jnp.maximum(
        jnp.dot(g1, sw2[...], preferred_element_type=jnp.float32) + sb2[...],
        0.0)
    o_ref[...] = (
        jnp.dot(g2, sw3[...], preferred_element_type=jnp.float32) + sb3[...])


def _edge_mlp(yai, aj, esi, esj, p, be=2048):
    full = lambda shape: pl.BlockSpec(shape, lambda i: tuple(0 for _ in shape))
    eb = lambda i: (i, 0)
    return pl.pallas_call(
        _mlp_body,
        grid=(EP // be,),
        in_specs=[
            pl.BlockSpec((be, 128), eb), pl.BlockSpec((be, 128), eb),
            pl.BlockSpec((be, 128), eb), pl.BlockSpec((be, 128), eb),
            full((128, 64)), full((1, 64)), full((1, 64)), full((1, 1)),
            full((1, 128)), full((1, 128)), full((128, 64)), full((1, 64)),
            full((64, 512)), full((1, 512)),
        ],
        out_specs=pl.BlockSpec((be, 512), eb),
        out_shape=jax.ShapeDtypeStruct((EP, 512), jnp.float32),
    )(yai, aj, esi, esj, p['e_w2'], p['e_b2'].reshape(1, 64),
      p['e_w3'].reshape(1, 64), p['e_b3'].reshape(1, 1),
      p['s_row'].reshape(1, 128), p['s_b1'].reshape(1, 128), p['s_w2'],
      p['s_b2'].reshape(1, 64), p['s_w3'], p['s_b3'].reshape(1, 512))


def _worker_id():
    return lax.axis_index("s") * NC + lax.axis_index("c")


@functools.partial(
    pl.kernel, mesh=_mesh,
    compiler_params=pltpu.CompilerParams(use_tc_tiling_on_sc=False),
    out_type=(
        jax.ShapeDtypeStruct((NC, NP, 136), jnp.float32),
        jax.ShapeDtypeStruct((EP, 128), jnp.float32),
        jax.ShapeDtypeStruct((EP, 128), jnp.float32),
    ),
    scratch_types=[
        pltpu.VMEM((NBLK1, BE1), jnp.int32),
        pltpu.VMEM((NBLK1, BE1), jnp.int32),
        pltpu.VMEM((BE1, 136), jnp.float32),
        pltpu.VMEM((BE1, 128), jnp.float32),
        pltpu.VMEM((BE1, 128), jnp.float32),
        pltpu.VMEM_SHARED((NP, 136), jnp.float32),
        pltpu.SemaphoreType.DMA,
        pltpu.SemaphoreType.DMA,
        pltpu.SemaphoreType.DMA,
    ],
)
def _k1(tab_baug, tab_s, idxi_hbm, idxj_hbm, zeros_hbm,
        nbr_hbm, esi_hbm, esj_hbm,
        idxi_v, idxj_v, g_b, g_si, g_sj, acc_sh, sem1, sem2, sem3):
    cid = lax.axis_index("c")
    sid = lax.axis_index("s")
    wid = sid * NC + cid
    pltpu.sync_copy(idxi_hbm.at[wid], idxi_v)
    pltpu.sync_copy(idxj_hbm.at[wid], idxj_v)

    @pl.when(sid == 0)
    def _zero():
        pltpu.sync_copy(zeros_hbm, acc_sh)

    plsc.subcore_barrier()
    base = wid * EPW

    def body(b, carry):
        cp1 = pltpu.async_copy(tab_baug.at[idxj_v.at[b]], g_b, sem1)
        cp2 = pltpu.async_copy(tab_s.at[idxi_v.at[b]], g_si, sem2)
        cp3 = pltpu.async_copy(tab_s.at[idxj_v.at[b]], g_sj, sem3)
        cp1.wait()
        cp2.wait()
        cp3.wait()
        pltpu.sync_copy(g_si, esi_hbm.at[pl.ds(base + b * BE1, BE1)])
        pltpu.sync_copy(g_sj, esj_hbm.at[pl.ds(base + b * BE1, BE1)])
        pltpu.sync_copy(g_b, acc_sh.at[idxi_v.at[b]], add=True)
        return carry

    lax.fori_loop(0, NBLK1, body, 0)
    plsc.subcore_barrier()

    @pl.when(sid == 0)
    def _out():
        pltpu.sync_copy(acc_sh, nbr_hbm.at[cid])


@functools.partial(
    pl.kernel, mesh=_mesh,
    compiler_params=pltpu.CompilerParams(use_tc_tiling_on_sc=False),
    out_type=(
        jax.ShapeDtypeStruct((EP, 128), jnp.float32),
        jax.ShapeDtypeStruct((EP, 128), jnp.float32),
    ),
    scratch_types=[
        pltpu.VMEM((NBLK3, BE3), jnp.int32),
        pltpu.VMEM((NBLK3, BE3), jnp.int32),
        pltpu.VMEM((BE3, 128), jnp.float32),
        pltpu.VMEM((BE3, 128), jnp.float32),
        pltpu.SemaphoreType.DMA,
        pltpu.SemaphoreType.DMA,
    ],
)
def _k3(tab_ya, tab_a, idxi_hbm, idxj_hbm, eyai_hbm, eaj_hbm,
        idxi_v, idxj_v, g_y, g_a, sem1, sem2):
    wid = _worker_id()
    pltpu.sync_copy(idxi_hbm.at[wid], idxi_v)
    pltpu.sync_copy(idxj_hbm.at[wid], idxj_v)
    base = wid * EPW

    def body(b, carry):
        cp1 = pltpu.async_copy(tab_ya.at[idxi_v.at[b]], g_y, sem1)
        cp2 = pltpu.async_copy(tab_a.at[idxj_v.at[b]], g_a, sem2)
        cp1.wait()
        cp2.wait()
        pltpu.sync_copy(g_y, eyai_hbm.at[pl.ds(base + b * BE3, BE3)])
        pltpu.sync_copy(g_a, eaj_hbm.at[pl.ds(base + b * BE3, BE3)])
        return carry

    lax.fori_loop(0, NBLK3, body, 0)


@functools.partial(
    pl.kernel, mesh=_mesh,
    compiler_params=pltpu.CompilerParams(use_tc_tiling_on_sc=False),
    out_type=jax.ShapeDtypeStruct((8, NC, NP, 128), jnp.float32),
    scratch_types=[
        pltpu.VMEM((NBLK1, BE1), jnp.int32),
        pltpu.VMEM((NBLK1, BE1), jnp.int32),
        pltpu.VMEM((BE1, 64), jnp.float32),
        pltpu.VMEM((BE1, 64), jnp.float32),
        pltpu.VMEM((BE1, 128), jnp.float32),
        pltpu.VMEM((BE1, 128), jnp.float32),
        pltpu.VMEM((BE1, 64), jnp.float32),
        pltpu.VMEM((BE1, 64), jnp.float32),
        pltpu.VMEM((BE1, 64), jnp.float32),
        pltpu.VMEM_SHARED((NPA, 64), jnp.float32),
        pltpu.VMEM_SHARED((NPA, 64), jnp.float32),
        pltpu.SemaphoreType.DMA,
        pltpu.SemaphoreType.DMA,
        pltpu.SemaphoreType.DMA,
        pltpu.SemaphoreType.DMA,
        pltpu.SemaphoreType.DMA,
        pltpu.SemaphoreType.DMA,
    ],
)
def _k5(tab_q, tab_kv, sz_hbm, idxi_hbm, idxj_hbm, zeros64_hbm, nd_hbm,
        idxi_v, idxj_v, q0, q1, kv0, kv1, sz0, sz1, m_v,
        acc_e, acc_m, sq0, sq1, sk0, sk1, sz0s, sz1s):
    cid = lax.axis_index("c")
    sid = lax.axis_index("s")
    wid = sid * NC + cid
    pltpu.sync_copy(idxi_hbm.at[wid], idxi_v)
    pltpu.sync_copy(idxj_hbm.at[wid], idxj_v)
    base = wid * EPW
    rings = ((q0, kv0, sz0, sq0, sk0, sz0s), (q1, kv1, sz1, sq1, sk1, sz1s))

    for c in range(8):
        def _issue(b, ring):
            qb, kvb, szb, s1, s2, s3 = ring
            pltpu.async_copy(tab_q.at[c].at[idxi_v.at[b]], qb, s1)
            pltpu.async_copy(tab_kv.at[c].at[idxj_v.at[b]], kvb, s2)
            pltpu.async_copy(
                sz_hbm.at[pl.ds(base + b * BE1, BE1), pl.ds(c * 64, 64)],
                szb, s3)

        def _wait(b, ring):
            qb, kvb, szb, s1, s2, s3 = ring
            pltpu.make_async_copy(tab_q.at[c].at[idxi_v.at[b]], qb, s1).wait()
            pltpu.make_async_copy(
                tab_kv.at[c].at[idxj_v.at[b]], kvb, s2).wait()
            pltpu.make_async_copy(
                sz_hbm.at[pl.ds(base + b * BE1, BE1), pl.ds(c * 64, 64)],
                szb, s3).wait()

        def _compute_scatter(b, ring):
            qb, kvb, szb = ring[0], ring[1], ring[2]

            def ebody(t2, c2):
                for dt in range(2):
                    t = t2 * 2 + dt
                    for r in range(4):
                        sl = pl.ds(r * 16, 16)
                        sl2 = pl.ds(64 + r * 16, 16)
                        e = jnp.exp(qb[t, sl] * kvb[t, sl])
                        qb[t, sl] = e
                        m_v[t, sl] = e * kvb[t, sl2] * szb[t, sl]
                return c2

            lax.fori_loop(0, BE1 // 2, ebody, 0)
            pltpu.sync_copy(qb, acc_e.at[idxi_v.at[b]], add=True)
            pltpu.sync_copy(m_v, acc_m.at[idxi_v.at[b]], add=True)

        @pl.when(sid == 0)
        def _zero():
            pltpu.sync_copy(zeros64_hbm, acc_e)
            pltpu.sync_copy(zeros64_hbm, acc_m)

        plsc.subcore_barrier()
        _issue(0, rings[0])

        def body2(h, carry):
            b0 = 2 * h
            b1 = 2 * h + 1
            _issue(b1, rings[1])
            _wait(b0, rings[0])
            _compute_scatter(b0, rings[0])

            @pl.when(b1 + 1 < NBLK1)
            def _next():
                _issue(b1 + 1, rings[0])

            _wait(b1, rings[1])
            _compute_scatter(b1, rings[1])
            return carry

        lax.fori_loop(0, NBLK1 // 2, body2, 0)
        plsc.subcore_barrier()

        @pl.when(sid == 0)
        def _out():
            pltpu.sync_copy(
                acc_e, nd_hbm.at[c].at[cid].at[pl.ds(0, NPA), pl.ds(0, 64)])
            pltpu.sync_copy(
                acc_m, nd_hbm.at[c].at[cid].at[pl.ds(0, NPA), pl.ds(64, 64)])


def _k2_body(ta, tb, nbr, eb1, ya_ref, invd_ref):
    nd = nbr[...]
    nb = nd[0, :, :128] + nd[1, :, :128]
    cnt = nd[0, :, 128] + nd[1, :, 128]
    maskc = (cnt > 0).astype(jnp.float32)
    invd = 1.0 / jnp.maximum(cnt, 1.0)
    ya_ref[...] = (ta[...] + 0.5 * maskc[:, None] * tb[...]
                   + 0.5 * invd[:, None] * nb + eb1[...])
    invd_ref[...] = jnp.broadcast_to(invd[:, None], invd_ref.shape)


def _node_combine(tab_a, tab_b, nbracc, e_b1, bn=2048):
    return pl.pallas_call(
        _k2_body,
        grid=(NP // bn,),
        in_specs=[
            pl.BlockSpec((bn, 128), lambda i: (i, 0)),
            pl.BlockSpec((bn, 128), lambda i: (i, 0)),
            pl.BlockSpec((NC, bn, 136), lambda i: (0, i, 0)),
            pl.BlockSpec((1, 128), lambda i: (0, 0)),
        ],
        out_specs=[
            pl.BlockSpec((bn, 128), lambda i: (i, 0)),
            pl.BlockSpec((bn, 64), lambda i: (i, 0)),
        ],
        out_shape=[
            jax.ShapeDtypeStruct((NP, 128), jnp.float32),
            jax.ShapeDtypeStruct((NP, 64), jnp.float32),
        ],
    )(tab_a, tab_b, nbracc, e_b1.reshape(1, 128))


def _k6_body(xw, nd_ref, invd, o_ref):
    nd = nd_ref[...]
    e = nd[:, 0, :, :64] + nd[:, 1, :, :64]
    m = nd[:, 0, :, 64:] + nd[:, 1, :, 64:]
    agg = m / (e + 1e-9)
    agg128 = jnp.concatenate([agg[0], agg[1]], axis=1)
    iv = invd[...]
    iv128 = jnp.concatenate([iv, iv], axis=1)
    o_ref[...] = jnp.maximum(
        jnp.maximum(xw[...], 0.0) + agg128 * iv128 * BN_C, 0.0)


def _finalize(xw, numden, invd, bn=2048):
    return pl.pallas_call(
        _k6_body,
        grid=(NP // bn, 4),
        in_specs=[
            pl.BlockSpec((bn, 128), lambda i, c: (i, c)),
            pl.BlockSpec((2, NC, bn, 128), lambda i, c: (c, 0, i, 0)),
            pl.BlockSpec((bn, 64), lambda i, c: (i, 0)),
        ],
        out_specs=pl.BlockSpec((bn, 128), lambda i, c: (i, c)),
        out_shape=jax.ShapeDtypeStruct((NP, 512), jnp.float32),
    )(xw, numden, invd)


def _prep_layer_params(p, din):
    inv = 1.0 / np.sqrt(float(CH))
    wtab = jnp.concatenate([
        0.5 * p['e_w1'][:din], p['e_w1'][din:], 0.5 * p['s_w1'][:din],
        p['wint_w'],
    ], axis=1)
    btab = jnp.concatenate([
        jnp.zeros((384,), jnp.float32), p['wint_b'],
    ])
    wq = p['q_w'] * inv
    bq = p['q_b'] * inv
    wkv = jnp.concatenate([
        p['k_w'].reshape(din, 8, 64), p['v_w'].reshape(din, 8, 64),
    ], axis=2).reshape(din, 1024)
    bkv = jnp.concatenate([
        p['k_b'].reshape(8, 64), p['v_b'].reshape(8, 64),
    ], axis=1).reshape(1024)
    return {
        'wtab': wtab, 'btab': btab, 'wq': wq, 'bq': bq,
        'wkv': wkv, 'bkv': bkv,
        'e_b1': p['e_b1'], 'e_w2': p['e_w2'], 'e_b2': p['e_b2'],
        'e_w3': p['e_w3'][:, 0], 'e_b3': p['e_b3'],
        's_row': p['s_w1'][din], 's_b1': p['s_b1'],
        's_w2': p['s_w2'], 's_b2': p['s_b2'],
        's_w3': p['s_w3'], 's_b3': p['s_b3'],
    }


def _layer(xpad, pp, idx3a, idx3b, zeros136, zeros64):
    idx_i3a, idx_j3a = idx3a
    idx_i3b, idx_j3b = idx3b
    proj = _mm(xpad, pp['wtab'], pp['btab'], bd=896)
    tab_a = proj[:, 0:128]
    tab_b = proj[:, 128:256]
    tab_s = proj[:, 256:384]
    xw = proj[:, 384:896]

    tab_baug = jnp.concatenate([
        tab_b,
        jnp.ones((NP, 1), jnp.float32),
        jnp.zeros((NP, 7), jnp.float32),
    ], axis=1)

    nbracc, esi, esj = _k1(tab_baug, tab_s, idx_i3a, idx_j3a, zeros136)
    ya, invd = _node_combine(tab_a, tab_b, nbracc, pp['e_b1'])
    eyai, eaj = _k3(ya, tab_a, idx_i3b, idx_j3b)
    sz = _edge_mlp(eyai, eaj, esi, esj, pp)

    tab_q = _mm_split(xpad, pp['wq'], pp['bq'], 64)
    tab_kv = _mm_split(xpad, pp['wkv'], pp['bkv'], 128)
    numden = _k5(tab_q, tab_kv, sz, idx_i3a, idx_j3a, zeros64)
    return _finalize(xw, numden, invd)


def kernel(x, edge_index, params):
    npad = EP - edge_index.shape[1]
    idx_i = jnp.concatenate(
        [edge_index[0], jnp.full((npad,), N_NODES, jnp.int32)])
    idx_j = jnp.concatenate(
        [edge_index[1], jnp.full((npad,), N_NODES, jnp.int32)])
    idx_i3a = idx_i.reshape(NW, NBLK1, BE1)
    idx_j3a = idx_j.reshape(NW, NBLK1, BE1)
    idx_i3b = idx_i.reshape(NW, NBLK3, BE3)
    idx_j3b = idx_j.reshape(NW, NBLK3, BE3)
    xpad = jnp.zeros((NP, x.shape[1]), jnp.float32).at[:N_NODES].set(x)

    zeros136 = jnp.zeros((NP, 136), jnp.float32)
    zeros64 = jnp.zeros((NPA, 64), jnp.float32)

    p1 = _prep_layer_params(params['layer1'], 256)
    p2 = _prep_layer_params(params['layer2'], 512)
    ia = (idx_i3a, idx_j3a)
    ib = (idx_i3b, idx_j3b)
    h = _layer(xpad, p1, ia, ib, zeros136, zeros64)
    h = _layer(h, p2, ia, ib, zeros136, zeros64)
    return h[:N_NODES]
